# baseline (speedup 1.0000x reference)
import functools

import jax
import jax.numpy as jnp
from jax import lax
from jax.experimental import pallas as pl
from jax.experimental.pallas import tpu as pltpu
from jax.experimental.pallas import tpu_sc as plsc

M = 1000000
D = 32
B = 16384
NC = 2
NS = 16
L = 16
NW = NC * NS
CC = 512
CCL2 = 9
NB = 4
NCHUNK = (M + CC - 1) // CC
LASTW = M - (NCHUNK - 1) * CC
PARTW = 0
TAILW = LASTW - PARTW
TAIL0 = M - TAILW
KMAX = (NCHUNK + NW - 1) // NW
GROUPS = B // L
SB = 32
CAP = CC + SB + L

_mesh = plsc.VectorSubcoreMesh(core_axis_name="c", subcore_axis_name="s")


@functools.partial(
    pl.kernel,
    out_type=(
        jax.ShapeDtypeStruct((D, M), jnp.float32),
        jax.ShapeDtypeStruct((M,), jnp.int32),
        jax.ShapeDtypeStruct((TAILW, D), jnp.float32),
    ),
    mesh=_mesh,
    scratch_types=[
        pltpu.VMEM((B,), jnp.int32),
        pltpu.VMEM((KMAX * CC,), jnp.int32),
        pltpu.VMEM((CAP,), jnp.int32),
        pltpu.VMEM((CAP,), jnp.int32),
        pltpu.VMEM((SB, 128), jnp.float32),
        pltpu.VMEM((SB,), jnp.int32),
        pltpu.VMEM((D, CC), jnp.float32),
        pltpu.VMEM((D, CC), jnp.float32),
        pltpu.VMEM((D, CC), jnp.float32),
        pltpu.VMEM((D, CC), jnp.float32),
        pltpu.VMEM((CC,), jnp.int32),
        pltpu.VMEM((CC,), jnp.int32),
        pltpu.VMEM((CC,), jnp.int32),
        pltpu.VMEM((CC,), jnp.int32),
        pltpu.VMEM((TAILW, D), jnp.float32),
        pltpu.SemaphoreType.DMA,
        pltpu.SemaphoreType.DMA,
        pltpu.SemaphoreType.DMA,
        pltpu.SemaphoreType.DMA,
        pltpu.SemaphoreType.DMA,
        pltpu.SemaphoreType.DMA,
        pltpu.SemaphoreType.DMA,
        pltpu.SemaphoreType.DMA,
        pltpu.SemaphoreType.DMA,
    ],
    compiler_params=pltpu.CompilerParams(needs_layout_passes=False),
)
def _replay_buffer_sc(memt_hbm, lab_hbm, idx_hbm, valp_hbm, nl_hbm, tin_hbm,
                      out_memt, out_lab, out_tail,
                      idx_v, stamp_v, cpos_v, ccol_v, rows_v, labv_v,
                      chunk0_v, chunk1_v, chunk2_v, chunk3_v,
                      labc0_v, labc1_v, labc2_v, labc3_v, tailb_v,
                      isem0, isem1, isem2, isem3,
                      osem0, osem1, osem2, osem3, gsem):
    c = lax.axis_index("c")
    s = lax.axis_index("s")
    w = c * NS + s
    lane = lax.iota(jnp.int32, L)
    neg1 = jnp.full((L,), -1, jnp.int32)
    cbufs = (chunk0_v, chunk1_v, chunk2_v, chunk3_v)
    lbufs = (labc0_v, labc1_v, labc2_v, labc3_v)
    isems = (isem0, isem1, isem2, isem3)
    osems = (osem0, osem1, osem2, osem3)

    nck = (NCHUNK - 1 - w) // NW + 1

    def in_descs(k, b):
        ck = w + k * NW
        col0 = pl.multiple_of(ck * CC, 128)
        full = [
            pltpu.make_async_copy(memt_hbm.at[:, pl.ds(col0, CC)],
                                  cbufs[b], isems[b]),
            pltpu.make_async_copy(lab_hbm.at[pl.ds(col0, CC)],
                                  lbufs[b], isems[b]),
        ]
        part = ([pltpu.make_async_copy(memt_hbm.at[:, pl.ds(col0, PARTW)],
                                       cbufs[b].at[:, pl.ds(0, PARTW)],
                                       isems[b])] if PARTW else []) + [
            pltpu.make_async_copy(lab_hbm.at[pl.ds(col0, LASTW)],
                                  lbufs[b].at[pl.ds(0, LASTW)], isems[b]),
            pltpu.make_async_copy(tin_hbm, tailb_v, isems[b]),
        ]
        return ck == NCHUNK - 1, full, part

    def out_descs(k, b):
        ck = w + k * NW
        col0 = pl.multiple_of(ck * CC, 128)
        full = [
            pltpu.make_async_copy(cbufs[b],
                                  out_memt.at[:, pl.ds(col0, CC)], osems[b]),
            pltpu.make_async_copy(lbufs[b],
                                  out_lab.at[pl.ds(col0, CC)], osems[b]),
        ]
        part = ([pltpu.make_async_copy(cbufs[b].at[:, pl.ds(0, PARTW)],
                                       out_memt.at[:, pl.ds(col0, PARTW)],
                                       osems[b])] if PARTW else []) + [
            pltpu.make_async_copy(lbufs[b].at[pl.ds(0, LASTW)],
                                  out_lab.at[pl.ds(col0, LASTW)], osems[b]),
            pltpu.make_async_copy(tailb_v, out_tail, osems[b]),
        ]
        return ck == NCHUNK - 1, full, part

    def dma_go(k, b, descs, start):
        is_part, full, part = descs(k, b)

        @pl.when(jnp.logical_not(is_part))
        def _f():
            for dsc in full:
                if start:
                    dsc.start()
                else:
                    dsc.wait()

        @pl.when(is_part)
        def _p():
            for dsc in part:
                if start:
                    dsc.start()
                else:
                    dsc.wait()

    def gather_block(base):
        pltpu.make_async_copy(valp_hbm.at[cpos_v.at[pl.ds(base, SB)]],
                              rows_v, gsem).start()
        pltpu.make_async_copy(nl_hbm.at[cpos_v.at[pl.ds(base, SB)]],
                              labv_v, gsem).start()

    def gather_wait(base):
        pltpu.make_async_copy(valp_hbm.at[cpos_v.at[pl.ds(base, SB)]],
                              rows_v, gsem).wait()
        pltpu.make_async_copy(nl_hbm.at[cpos_v.at[pl.ds(base, SB)]],
                              labv_v, gsem).wait()

    def scan_prefetch(k):
        def scan_body(g2, cnt):
            sv = stamp_v[pl.ds(k * CC + g2 * L, L)]
            vm = sv >= 0
            plsc.store_compressed(cpos_v.at[pl.ds(cnt, L)], sv, mask=vm)
            plsc.store_compressed(ccol_v.at[pl.ds(cnt, L)],
                                  g2 * L + lane, mask=vm)
            return cnt + jnp.max(plsc.all_reduce_population_count(vm))

        cnt = lax.fori_loop(0, CC // L, scan_body, 0)

        @pl.when(cnt > 0)
        def _pad_and_fetch():
            zero16 = jnp.zeros((L,), jnp.int32)
            for t in range(SB // L):
                pad16 = (w * SB + t * L + lane) & (B - 1)
                plsc.store_scatter(cpos_v, [cnt + t * L + lane], pad16)
                plsc.store_scatter(ccol_v, [cnt + t * L + lane], zero16)
            gather_block(0)

        return cnt

    def process(k, b, cnt):
        ck = w + k * NW
        is_part = ck == NCHUNK - 1
        cbuf, lbuf = cbufs[b], lbufs[b]

        @pl.when(cnt > 0)
        def _apply():
            nsb = (cnt + SB - 1) // SB

            def write_block(sb, carry):
                base = sb * SB
                gather_wait(base)
                rem = jnp.minimum(cnt - base, SB)
                ng = (rem + L - 1) // L

                def g_body(t, carry2):
                    r16 = t * L + lane
                    cv = ccol_v[pl.ds(base + t * L, L)]
                    valid = r16 < rem
                    lv = plsc.load_gather(labv_v, [r16])
                    plsc.store_scatter(lbuf, [cv], lv, mask=valid)
                    for d in range(D):
                        d16 = jnp.full((L,), d, jnp.int32)
                        vals = plsc.load_gather(rows_v, [r16, d16])

                        @pl.when(jnp.logical_not(is_part))
                        def _wmain(d16=d16, cv=cv, vals=vals, valid=valid):
                            plsc.store_scatter(cbuf, [d16, cv], vals,
                                               mask=valid)

                        @pl.when(is_part)
                        def _wpart(d16=d16, cv=cv, vals=vals, valid=valid):
                            plsc.store_scatter(cbuf, [d16, cv], vals,
                                               mask=valid & (cv < PARTW))
                            cvt = jnp.maximum(cv - PARTW, 0)
                            plsc.store_scatter(tailb_v, [cvt, d16], vals,
                                               mask=valid & (cv >= PARTW))

                    return carry2

                lax.fori_loop(0, ng, g_body, 0)

                @pl.when(sb + 1 < nsb)
                def _next():
                    gather_block((sb + 1) * SB)

                return carry

            lax.fori_loop(0, nsb, write_block, 0)

    for j in range(NB - 1):
        dma_go(j, j, in_descs, True)

    def clr_body(i, carry):
        stamp_v[pl.ds(i * L, L)] = neg1
        return carry

    lax.fori_loop(0, KMAX * CC // L, clr_body, 0)

    pltpu.sync_copy(idx_hbm, idx_v)

    def stamp_body(g, carry):
        iv = idx_v[pl.ds(g * L, L)]
        ckv = lax.shift_right_logical(iv, CCL2)
        m = (ckv & (NW - 1)) == w
        local = (lax.shift_right_logical(ckv, 5) * CC) | (iv & (CC - 1))
        local = jnp.where(m, local, 0)
        plsc.store_scatter(stamp_v, [local], g * L + lane, mask=m)
        return carry

    lax.fori_loop(0, GROUPS, stamp_body, 0)

    npasses = (nck + NB - 1) // NB

    def pass_body(p, carry):
        for b in range(NB):
            kk = NB * p + b

            @pl.when(kk < nck)
            def _step(kk=kk, b=b):
                @pl.when(kk + NB - 1 < nck)
                def _refill():
                    @pl.when(kk >= 1)
                    def _drain_prev():
                        dma_go(kk - 1, (b - 1) % NB, out_descs, False)

                    dma_go(kk + NB - 1, (b - 1) % NB, in_descs, True)

                cnt = scan_prefetch(kk)
                dma_go(kk, b, in_descs, False)
                process(kk, b, cnt)
                dma_go(kk, b, out_descs, True)

        return carry

    lax.fori_loop(0, npasses, pass_body, 0)

    for b in range(NB):
        @pl.when((nck - 1 - ((nck - 1 - b) % NB)) >= 0)
        def _drain_last(b=b):
            dma_go(nck - 1 - ((nck - 1 - b) % NB), b, out_descs, False)


def kernel(mem, labels_buf, idx, val, new_labels):
    valp = jnp.zeros((B, 128), jnp.float32).at[:, :D].set(val)
    tail_in = lax.slice(mem, (TAIL0, 0), (M, D))
    out_memt, out_lab, out_tail = _replay_buffer_sc(
        mem.T,
        labels_buf.astype(jnp.int32),
        idx.astype(jnp.int32),
        valp,
        new_labels.astype(jnp.int32),
        tail_in,
    )
    new_mem = out_memt.T.at[TAIL0:, :].set(out_tail)
    return new_mem, out_lab.astype(labels_buf.dtype)


# speedup vs baseline: 1.3695x; 1.3695x over previous
import functools

import jax
import jax.numpy as jnp
from jax import lax
from jax.experimental import pallas as pl
from jax.experimental.pallas import tpu as pltpu
from jax.experimental.pallas import tpu_sc as plsc

M = 1000000
D = 32
B = 16384
NC = 2
NS = 16
L = 16
NW = NC * NS
CC = 1024
NCHUNK = (M + CC - 1) // CC
LASTW = M - (NCHUNK - 1) * CC
PARTW = 512
TAILW = LASTW - PARTW
TAIL0 = M - TAILW
KMAX = (NCHUNK + NW - 1) // NW
GROUPS = B // L
SB = 32
CAP = CC + SB + L

_mesh = plsc.VectorSubcoreMesh(core_axis_name="c", subcore_axis_name="s")


@functools.partial(
    pl.kernel,
    out_type=(
        jax.ShapeDtypeStruct((D, M), jnp.float32),
        jax.ShapeDtypeStruct((M,), jnp.int32),
        jax.ShapeDtypeStruct((TAILW, D), jnp.float32),
    ),
    mesh=_mesh,
    scratch_types=[
        pltpu.VMEM((B,), jnp.int32),
        pltpu.VMEM((KMAX * CC,), jnp.int32),
        pltpu.VMEM((CAP,), jnp.int32),
        pltpu.VMEM((CAP,), jnp.int32),
        pltpu.VMEM((SB, 128), jnp.float32),
        pltpu.VMEM((SB,), jnp.int32),
        pltpu.VMEM((D, CC), jnp.float32),
        pltpu.VMEM((D, CC), jnp.float32),
        pltpu.VMEM((CC,), jnp.int32),
        pltpu.VMEM((CC,), jnp.int32),
        pltpu.VMEM((TAILW, D), jnp.float32),
        pltpu.SemaphoreType.DMA,
        pltpu.SemaphoreType.DMA,
        pltpu.SemaphoreType.DMA,
        pltpu.SemaphoreType.DMA,
        pltpu.SemaphoreType.DMA,
    ],
    compiler_params=pltpu.CompilerParams(needs_layout_passes=False),
)
def _replay_buffer_sc(memt_hbm, lab_hbm, idx_hbm, valp_hbm, nl_hbm, tin_hbm,
                      out_memt, out_lab, out_tail,
                      idx_v, stamp_v, cpos_v, ccol_v, rows_v, labv_v,
                      chunk0_v, chunk1_v, labc0_v, labc1_v, tailb_v,
                      isem0, isem1, osem0, osem1, gsem):
    c = lax.axis_index("c")
    s = lax.axis_index("s")
    w = c * NS + s
    lane = lax.iota(jnp.int32, L)
    neg1 = jnp.full((L,), -1, jnp.int32)
    cbufs = (chunk0_v, chunk1_v)
    lbufs = (labc0_v, labc1_v)
    isems = (isem0, isem1)
    osems = (osem0, osem1)

    nck = (NCHUNK - 1 - w) // NW + 1

    def in_descs(k, b):
        ck = w + k * NW
        col0 = pl.multiple_of(ck * CC, 128)
        full = [
            pltpu.make_async_copy(memt_hbm.at[:, pl.ds(col0, CC)],
                                  cbufs[b], isems[b]),
            pltpu.make_async_copy(lab_hbm.at[pl.ds(col0, CC)],
                                  lbufs[b], isems[b]),
        ]
        part = [
            pltpu.make_async_copy(memt_hbm.at[:, pl.ds(col0, PARTW)],
                                  cbufs[b].at[:, pl.ds(0, PARTW)], isems[b]),
            pltpu.make_async_copy(lab_hbm.at[pl.ds(col0, LASTW)],
                                  lbufs[b].at[pl.ds(0, LASTW)], isems[b]),
            pltpu.make_async_copy(tin_hbm, tailb_v, isems[b]),
        ]
        return ck == NCHUNK - 1, full, part

    def out_descs(k, b):
        ck = w + k * NW
        col0 = pl.multiple_of(ck * CC, 128)
        full = [
            pltpu.make_async_copy(cbufs[b],
                                  out_memt.at[:, pl.ds(col0, CC)], osems[b]),
            pltpu.make_async_copy(lbufs[b],
                                  out_lab.at[pl.ds(col0, CC)], osems[b]),
        ]
        part = [
            pltpu.make_async_copy(cbufs[b].at[:, pl.ds(0, PARTW)],
                                  out_memt.at[:, pl.ds(col0, PARTW)], osems[b]),
            pltpu.make_async_copy(lbufs[b].at[pl.ds(0, LASTW)],
                                  out_lab.at[pl.ds(col0, LASTW)], osems[b]),
            pltpu.make_async_copy(tailb_v, out_tail, osems[b]),
        ]
        return ck == NCHUNK - 1, full, part

    def dma_go(k, b, descs, start):
        is_part, full, part = descs(k, b)

        @pl.when(jnp.logical_not(is_part))
        def _f():
            for dsc in full:
                if start:
                    dsc.start()
                else:
                    dsc.wait()

        @pl.when(is_part)
        def _p():
            for dsc in part:
                if start:
                    dsc.start()
                else:
                    dsc.wait()

    def gather_block(base):
        pltpu.make_async_copy(valp_hbm.at[cpos_v.at[pl.ds(base, SB)]],
                              rows_v, gsem).start()
        pltpu.make_async_copy(nl_hbm.at[cpos_v.at[pl.ds(base, SB)]],
                              labv_v, gsem).start()

    def gather_wait(base):
        pltpu.make_async_copy(valp_hbm.at[cpos_v.at[pl.ds(base, SB)]],
                              rows_v, gsem).wait()
        pltpu.make_async_copy(nl_hbm.at[cpos_v.at[pl.ds(base, SB)]],
                              labv_v, gsem).wait()

    def scan_prefetch(k):
        def scan_body(g2, cnt):
            sv = stamp_v[pl.ds(k * CC + g2 * L, L)]
            vm = sv >= 0
            plsc.store_compressed(cpos_v.at[pl.ds(cnt, L)], sv, mask=vm)
            plsc.store_compressed(ccol_v.at[pl.ds(cnt, L)],
                                  g2 * L + lane, mask=vm)
            return cnt + jnp.max(plsc.all_reduce_population_count(vm))

        cnt = lax.fori_loop(0, CC // L, scan_body, 0)

        @pl.when(cnt > 0)
        def _pad_and_fetch():
            zero16 = jnp.zeros((L,), jnp.int32)
            for t in range(SB // L):
                pad16 = (w * SB + t * L + lane) & (B - 1)
                plsc.store_scatter(cpos_v, [cnt + t * L + lane], pad16)
                plsc.store_scatter(ccol_v, [cnt + t * L + lane], zero16)
            gather_block(0)

        return cnt

    def process(k, b, cnt):
        ck = w + k * NW
        is_part = ck == NCHUNK - 1
        cbuf, lbuf = cbufs[b], lbufs[b]

        @pl.when(cnt > 0)
        def _apply():
            nsb = (cnt + SB - 1) // SB

            def write_block(sb, carry):
                base = sb * SB
                gather_wait(base)
                rem = jnp.minimum(cnt - base, SB)
                ng = (rem + L - 1) // L

                def g_body(t, carry2):
                    r16 = t * L + lane
                    cv = ccol_v[pl.ds(base + t * L, L)]
                    valid = r16 < rem
                    lv = plsc.load_gather(labv_v, [r16])
                    plsc.store_scatter(lbuf, [cv], lv, mask=valid)
                    for d in range(D):
                        d16 = jnp.full((L,), d, jnp.int32)
                        vals = plsc.load_gather(rows_v, [r16, d16])

                        @pl.when(jnp.logical_not(is_part))
                        def _wmain(d16=d16, cv=cv, vals=vals, valid=valid):
                            plsc.store_scatter(cbuf, [d16, cv], vals,
                                               mask=valid)

                        @pl.when(is_part)
                        def _wpart(d16=d16, cv=cv, vals=vals, valid=valid):
                            plsc.store_scatter(cbuf, [d16, cv], vals,
                                               mask=valid & (cv < PARTW))
                            cvt = jnp.maximum(cv - PARTW, 0)
                            plsc.store_scatter(tailb_v, [cvt, d16], vals,
                                               mask=valid & (cv >= PARTW))

                    return carry2

                lax.fori_loop(0, ng, g_body, 0)

                @pl.when(sb + 1 < nsb)
                def _next():
                    gather_block((sb + 1) * SB)

                return carry

            lax.fori_loop(0, nsb, write_block, 0)

    dma_go(0, 0, in_descs, True)

    @pl.when(nck > 1)
    def _pre1():
        dma_go(1, 1, in_descs, True)

    def clr_body(i, carry):
        stamp_v[pl.ds(i * L, L)] = neg1
        return carry

    lax.fori_loop(0, KMAX * CC // L, clr_body, 0)

    pltpu.sync_copy(idx_hbm, idx_v)

    def stamp_body(g, carry):
        iv = idx_v[pl.ds(g * L, L)]
        ckv = lax.shift_right_logical(iv, 10)
        m = (ckv & (NW - 1)) == w
        local = (lax.shift_right_logical(ckv, 5) * CC) | (iv & (CC - 1))
        local = jnp.where(m, local, 0)
        plsc.store_scatter(stamp_v, [local], g * L + lane, mask=m)
        return carry

    lax.fori_loop(0, GROUPS, stamp_body, 0)

    npairs = (nck + 1) // 2

    def pair_body(p, carry):
        for b in range(2):
            kk = 2 * p + b

            @pl.when(kk < nck)
            def _step(kk=kk, b=b):
                cnt = scan_prefetch(kk)

                @pl.when(kk >= 1)
                def _drain_prev():
                    dma_go(kk - 1, 1 - b, out_descs, False)

                @pl.when((kk >= 1) & (kk + 1 < nck))
                def _prefetch():
                    dma_go(kk + 1, 1 - b, in_descs, True)

                dma_go(kk, b, in_descs, False)
                process(kk, b, cnt)
                dma_go(kk, b, out_descs, True)

        return carry

    lax.fori_loop(0, npairs, pair_body, 0)

    for b in range(2):
        @pl.when((nck - 1) % 2 == b)
        def _drain_last(b=b):
            dma_go(nck - 1, b, out_descs, False)


def kernel(mem, labels_buf, idx, val, new_labels):
    valp = jnp.zeros((B, 128), jnp.float32).at[:, :D].set(val)
    tail_in = lax.slice(mem, (TAIL0, 0), (M, D))
    out_memt, out_lab, out_tail = _replay_buffer_sc(
        mem.T,
        labels_buf.astype(jnp.int32),
        idx.astype(jnp.int32),
        valp,
        new_labels.astype(jnp.int32),
        tail_in,
    )
    new_mem = out_memt.T.at[TAIL0:, :].set(out_tail)
    return new_mem, out_lab.astype(labels_buf.dtype)
